# baseline (speedup 1.0000x reference)
import functools

import jax
import jax.numpy as jnp
from jax import lax
from jax.experimental import pallas as pl
from jax.experimental.pallas import tpu as pltpu
from jax.experimental.pallas import tpu_sc as plsc

B = 8
HS = 256
H = 512
W = 512
HW = H * W
NITER = 5
P = NITER + 1
HP = H + 2 * P

NSHARD = 8
SHARD = HW // NSHARD
SLAB = 8192
BIG = 2 ** 30

_G = [0.27406862, 0.45186276, 0.27406862]


def _field_body(sx_ref, sy_ref, f_ref, dx_ref, dy_ref):
    sx = sx_ref[0]
    sy = sy_ref[0]
    col = lax.broadcasted_iota(jnp.int32, (HS, HS), 1).astype(jnp.float32)
    row = lax.broadcasted_iota(jnp.int32, (HS, HS), 0).astype(jnp.float32)
    dxs = sx - (col * (2.0 / (HS - 1)) - 1.0)
    dys = sy - (row * (2.0 / (HS - 1)) - 1.0)

    def colsep(a):
        left = jnp.concatenate([a[:, :1], a[:, :-1]], axis=1)
        right = jnp.concatenate([a[:, 1:], a[:, -1:]], axis=1)
        return 0.25 * left + 0.75 * a, 0.75 * a + 0.25 * right

    def rowsep(a):
        up = jnp.concatenate([a[:1], a[:-1]], axis=0)
        dn = jnp.concatenate([a[1:], a[-1:]], axis=0)
        return 0.25 * up + 0.75 * a, 0.75 * a + 0.25 * dn

    def planes(a):
        ce, co = colsep(a)
        p00, p10 = rowsep(ce)
        p01, p11 = rowsep(co)
        return (p00, p01, p10, p11)

    px = planes(dxs)
    py = planes(dys)
    colu = lax.broadcasted_iota(jnp.int32, (HS, HS), 1)
    rowu = lax.broadcasted_iota(jnp.int32, (HS, HS), 0)
    fs = []
    dxo = []
    dyo = []
    for plane in range(4):
        rp, cp = plane >> 1, plane & 1
        dx = px[plane] * (W / 2.0)
        dy = py[plane] * (H / 2.0)
        xpos = (2 * colu + cp).astype(jnp.float32)
        ypos = (2 * rowu + rp).astype(jnp.float32)
        xg = jnp.round(xpos + dx).astype(jnp.int32)
        yg = jnp.round(ypos + dy).astype(jnp.int32)
        oob = (xg < 0) | (yg < 0) | (xg > W - 1) | (yg > H - 1)
        t = jnp.where(oob, -1, yg * W + xg)
        fs.append(t)
        dxo.append(dx)
        dyo.append(dy)
    f_ref[0] = jnp.stack(fs, axis=0)
    dx_ref[0] = jnp.stack(dxo, axis=0)
    dy_ref[0] = jnp.stack(dyo, axis=0)


def _stage_a(sx, sy):
    shp = jax.ShapeDtypeStruct((B, 4, HS, HS), jnp.float32)
    return pl.pallas_call(
        _field_body,
        grid=(B,),
        in_specs=[pl.BlockSpec((1, HS, HS), lambda b: (b, 0, 0))] * 2,
        out_specs=[pl.BlockSpec((1, 4, HS, HS), lambda b: (b, 0, 0, 0))] * 3,
        out_shape=[jax.ShapeDtypeStruct((B, 4, HS, HS), jnp.int32), shp, shp],
    )(sx, sy)


def _scatter_body(f_hbm, dx_hbm, dy_hbm, win_out, vx_out, vy_out,
                  win_v, vx_v, vy_v, f_s, dx_s, dy_s):
    nc = 2
    wid = lax.axis_index("s") * nc + lax.axis_index("c")
    lane = lax.iota(jnp.int32, 16)
    big = jnp.full((16,), BIG, jnp.int32)
    zero = jnp.zeros((16,), jnp.float32)

    for rnd in range(2):
        task = wid + 32 * rnd
        b = task // NSHARD
        shard = task % NSHARD
        lo = shard * SHARD

        @pl.loop(0, SHARD // 16)
        def _init(j):
            off = j * 16
            win_v[pl.ds(off, 16)] = big
            vx_v[pl.ds(off, 16)] = zero
            vy_v[pl.ds(off, 16)] = zero

        for slab in range(HW // SLAB):
            base = slab * SLAB
            pltpu.sync_copy(f_hbm.at[b, pl.ds(base, SLAB)], f_s)
            pltpu.sync_copy(dx_hbm.at[b, pl.ds(base, SLAB)], dx_s)
            pltpu.sync_copy(dy_hbm.at[b, pl.ds(base, SLAB)], dy_s)

            @pl.loop(0, SLAB // 16)
            def _vec(v):
                off = v * 16
                p0 = base + off
                t = f_s[pl.ds(off, 16)]
                cnt, _last = plsc.scan_count(t)
                keep = cnt == 1
                m = keep & (t >= lo) & (t < lo + SHARD)
                plane = p0 >> 16
                r = (p0 >> 8) & 255
                k0 = p0 & 255
                i_base = r * 1024 + (plane >> 1) * 512 + 2 * k0 + (plane & 1)
                i_vec = i_base + 2 * lane
                loc = jnp.clip(t - lo, 0, SHARD - 1)
                cur = plsc.load_gather(win_v, [loc], mask=m)
                wm = m & (i_vec < cur)
                plsc.store_scatter(win_v, [loc], i_vec, mask=wm)
                plsc.store_scatter(vx_v, [loc], -dx_s[pl.ds(off, 16)], mask=wm)
                plsc.store_scatter(vy_v, [loc], -dy_s[pl.ds(off, 16)], mask=wm)

        pltpu.sync_copy(win_v, win_out.at[b, shard])
        pltpu.sync_copy(vx_v, vx_out.at[b, shard])
        pltpu.sync_copy(vy_v, vy_out.at[b, shard])


def _stage_b(field, dx, dy):
    mesh = plsc.VectorSubcoreMesh(core_axis_name="c", subcore_axis_name="s")
    shp = jax.ShapeDtypeStruct((B, NSHARD, SHARD), jnp.float32)
    fn = functools.partial(
        pl.kernel,
        mesh=mesh,
        compiler_params=pltpu.CompilerParams(needs_layout_passes=False),
        out_type=[jax.ShapeDtypeStruct((B, NSHARD, SHARD), jnp.int32), shp, shp],
        scratch_types=[
            pltpu.VMEM((SHARD,), jnp.int32),
            pltpu.VMEM((SHARD,), jnp.float32),
            pltpu.VMEM((SHARD,), jnp.float32),
            pltpu.VMEM((SLAB,), jnp.int32),
            pltpu.VMEM((SLAB,), jnp.float32),
            pltpu.VMEM((SLAB,), jnp.float32),
        ],
    )(_scatter_body)
    return fn(field.reshape(B, HW), dx.reshape(B, HW), dy.reshape(B, HW))


def _pad2(a, val):
    cval = jnp.full((1, 1), val, a.dtype)
    rows = jnp.concatenate(
        [jnp.broadcast_to(cval, (P, a.shape[1])), a,
         jnp.broadcast_to(cval, (P, a.shape[1]))], axis=0)
    return jnp.concatenate(
        [jnp.broadcast_to(cval, (HP, P)), rows,
         jnp.broadcast_to(cval, (HP, P))], axis=1)


def _shift(a, dr, dc, fill):
    out = a
    if dr:
        z = jnp.full((abs(dr), a.shape[1]), fill, a.dtype)
        out = (jnp.concatenate([z, out[:-dr]], axis=0) if dr > 0
               else jnp.concatenate([out[-dr:], z], axis=0))
    if dc:
        z = jnp.full((a.shape[0], abs(dc)), fill, a.dtype)
        out = (jnp.concatenate([z, out[:, :-dc]], axis=1) if dc > 0
               else jnp.concatenate([out[:, -dc:], z], axis=1))
    return out


def _conv_sep(a):
    v = _G[0] * _shift(a, 1, 0, 0.0) + _G[1] * a + _G[2] * _shift(a, -1, 0, 0.0)
    return _G[0] * _shift(v, 0, 1, 0.0) + _G[1] * v + _G[2] * _shift(v, 0, -1, 0.0)


def _nbr(m):
    return (_shift(m, 1, 0, 0) | _shift(m, -1, 0, 0)
            | _shift(m, 0, 1, 0) | _shift(m, 0, -1, 0))


def _fill_body(win_ref, vx_ref, vy_ref, ox_ref, oy_ref):
    m = _pad2((win_ref[0] < BIG).astype(jnp.int32), 0)
    px = _pad2(vx_ref[0], 0.0)
    py = _pad2(vy_ref[0], 0.0)
    for _ in range(NITER):
        nm = (1 - m) & _nbr(m)
        cx = _conv_sep(px)
        cy = _conv_sep(py)
        cs = _conv_sep(m.astype(jnp.float32))
        hole = nm > 0
        denom = jnp.where(hole, cs, 1.0)
        px = jnp.where(hole, cx / denom, px)
        py = jnp.where(hole, cy / denom, py)
        m = m | nm
    for _ in range(NITER):
        m = m & (1 - (m & _nbr(1 - m)))
    keepm = m > 0
    px = jnp.where(keepm, px, 2.0 * W)[P:-P, P:-P]
    py = jnp.where(keepm, py, 2.0 * H)[P:-P, P:-P]
    gx = lax.broadcasted_iota(jnp.int32, (H, W), 1).astype(jnp.float32)
    gy = lax.broadcasted_iota(jnp.int32, (H, W), 0).astype(jnp.float32)
    ox_ref[0] = gx * (2.0 / (W - 1)) - 1.0 + px * (2.0 / W)
    oy_ref[0] = gy * (2.0 / (H - 1)) - 1.0 + py * (2.0 / H)


def _stage_c(win, vx, vy):
    shp = jax.ShapeDtypeStruct((B, H, W), jnp.float32)
    return pl.pallas_call(
        _fill_body,
        grid=(B,),
        in_specs=[pl.BlockSpec((1, H, W), lambda b: (b, 0, 0))] * 3,
        out_specs=[pl.BlockSpec((1, H, W), lambda b: (b, 0, 0))] * 2,
        out_shape=[shp, shp],
    )(win.reshape(B, H, W), vx.reshape(B, H, W), vy.reshape(B, H, W))


def kernel(src_grid):
    sx = src_grid[..., 0]
    sy = src_grid[..., 1]
    field, dx, dy = _stage_a(sx, sy)
    win, vx, vy = _stage_b(field, dx, dy)
    ox, oy = _stage_c(win, vx, vy)
    return jnp.stack([ox, oy], axis=-1)


# speedup vs baseline: 2.3518x; 2.3518x over previous
import functools

import jax
import jax.numpy as jnp
from jax import lax
from jax.experimental import pallas as pl
from jax.experimental.pallas import tpu as pltpu
from jax.experimental.pallas import tpu_sc as plsc

B = 8
HS = 256
H = 512
W = 512
HW = H * W
NITER = 5
P = NITER + 1
HP = H + 2 * P

NSHARD = 8
SHARD = HW // NSHARD
SLAB = 4096
BIG = 2 ** 30

_G = [0.27406862, 0.45186276, 0.27406862]


def _field_body(sx_ref, sy_ref, f_ref, dx_ref, dy_ref, mm_ref):
    sx = sx_ref[0]
    sy = sy_ref[0]
    col = lax.broadcasted_iota(jnp.int32, (HS, HS), 1).astype(jnp.float32)
    row = lax.broadcasted_iota(jnp.int32, (HS, HS), 0).astype(jnp.float32)
    dxs = sx - (col * (2.0 / (HS - 1)) - 1.0)
    dys = sy - (row * (2.0 / (HS - 1)) - 1.0)

    def colsep(a):
        left = jnp.concatenate([a[:, :1], a[:, :-1]], axis=1)
        right = jnp.concatenate([a[:, 1:], a[:, -1:]], axis=1)
        return 0.25 * left + 0.75 * a, 0.75 * a + 0.25 * right

    def rowsep(a):
        up = jnp.concatenate([a[:1], a[:-1]], axis=0)
        dn = jnp.concatenate([a[1:], a[-1:]], axis=0)
        return 0.25 * up + 0.75 * a, 0.75 * a + 0.25 * dn

    def planes(a):
        ce, co = colsep(a)
        p00, p10 = rowsep(ce)
        p01, p11 = rowsep(co)
        return (p00, p01, p10, p11)

    px = planes(dxs)
    py = planes(dys)
    colu = lax.broadcasted_iota(jnp.int32, (HS, HS), 1)
    rowu = lax.broadcasted_iota(jnp.int32, (HS, HS), 0)
    fs = []
    dxo = []
    dyo = []
    for plane in range(4):
        rp, cp = plane >> 1, plane & 1
        dx = px[plane] * (W / 2.0)
        dy = py[plane] * (H / 2.0)
        xpos = (2 * colu + cp).astype(jnp.float32)
        ypos = (2 * rowu + rp).astype(jnp.float32)
        xg = jnp.round(xpos + dx).astype(jnp.int32)
        yg = jnp.round(ypos + dy).astype(jnp.int32)
        oob = (xg < 0) | (yg < 0) | (xg > W - 1) | (yg > H - 1)
        t = jnp.where(oob, -1, yg * W + xg)
        fs.append(t)
        dxo.append(dx)
        dyo.append(dy)
    f_ref[0] = jnp.stack(fs, axis=0)
    dx_ref[0] = jnp.stack(dxo, axis=0)
    dy_ref[0] = jnp.stack(dyo, axis=0)
    mns = [jnp.min(jnp.where(t < 0, HW, t), axis=1) for t in fs]
    mxs = [jnp.max(t, axis=1) for t in fs]
    mm_ref[0] = jnp.stack(
        [jnp.concatenate(mns), jnp.concatenate(mxs)], axis=0)


def _stage_a(sx, sy):
    shp = jax.ShapeDtypeStruct((B, 4, HS, HS), jnp.float32)
    return pl.pallas_call(
        _field_body,
        grid=(B,),
        in_specs=[pl.BlockSpec((1, HS, HS), lambda b: (b, 0, 0))] * 2,
        out_specs=[pl.BlockSpec((1, 4, HS, HS), lambda b: (b, 0, 0, 0))] * 3
        + [pl.BlockSpec((1, 2, 4 * HS), lambda b: (b, 0, 0))],
        out_shape=[jax.ShapeDtypeStruct((B, 4, HS, HS), jnp.int32), shp, shp,
                   jax.ShapeDtypeStruct((B, 2, 4 * HS), jnp.int32)],
    )(sx, sy)


def _scatter_body(f_hbm, dx_hbm, dy_hbm, mm_hbm, win_out, vx_out, vy_out,
                  win_v, vx_v, vy_v, f_s, dx_s, dy_s, mn_s, mx_s, sem):
    nc = 2
    wid = lax.axis_index("s") * nc + lax.axis_index("c")
    lane = lax.iota(jnp.int32, 16)
    big = jnp.full((16,), BIG, jnp.int32)
    zero = jnp.zeros((16,), jnp.float32)

    for rnd in range(2):
        task = wid + 32 * rnd
        b = task // NSHARD
        shard = task % NSHARD
        lo = shard * SHARD

        @pl.loop(0, SHARD // 16)
        def _init(j):
            off = j * 16
            win_v[pl.ds(off, 16)] = big
            vx_v[pl.ds(off, 16)] = zero
            vy_v[pl.ds(off, 16)] = zero

        pltpu.sync_copy(mm_hbm.at[b, 0], mn_s)
        pltpu.sync_copy(mm_hbm.at[b, 1], mx_s)

        @pl.loop(0, HW // SLAB)
        def _slab(s):
            mn16 = mn_s[pl.ds(s * 16, 16)]
            mx16 = mx_s[pl.ds(s * 16, 16)]
            rel = (mx16 >= lo) & (mn16 < lo + SHARD)
            cnt = jnp.sum(rel.astype(jnp.int32))

            @pl.when(cnt > 0)
            def _():
                base = s * SLAB
                c1 = pltpu.async_copy(f_hbm.at[b, pl.ds(base, SLAB)], f_s, sem)
                c2 = pltpu.async_copy(dx_hbm.at[b, pl.ds(base, SLAB)], dx_s, sem)
                c3 = pltpu.async_copy(dy_hbm.at[b, pl.ds(base, SLAB)], dy_s, sem)
                c1.wait()
                c2.wait()
                c3.wait()

                @pl.loop(0, SLAB // 16)
                def _vec(v):
                    off = v * 16
                    p0 = base + off
                    t = f_s[pl.ds(off, 16)]
                    cnt_, _last = plsc.scan_count(t)
                    keep = cnt_ == 1
                    m = keep & (t >= lo) & (t < lo + SHARD)
                    plane = p0 >> 16
                    r = (p0 >> 8) & 255
                    k0 = p0 & 255
                    i_base = (r * 1024 + (plane >> 1) * 512 + 2 * k0
                              + (plane & 1))
                    i_vec = i_base + 2 * lane
                    loc = jnp.clip(t - lo, 0, SHARD - 1)
                    cur = plsc.load_gather(win_v, [loc], mask=m)
                    wm = m & (i_vec < cur)
                    plsc.store_scatter(win_v, [loc], i_vec, mask=wm)
                    plsc.store_scatter(vx_v, [loc], -dx_s[pl.ds(off, 16)],
                                       mask=wm)
                    plsc.store_scatter(vy_v, [loc], -dy_s[pl.ds(off, 16)],
                                       mask=wm)

        pltpu.sync_copy(win_v, win_out.at[b, shard])
        pltpu.sync_copy(vx_v, vx_out.at[b, shard])
        pltpu.sync_copy(vy_v, vy_out.at[b, shard])


def _stage_b(field, dx, dy, minmax):
    mesh = plsc.VectorSubcoreMesh(core_axis_name="c", subcore_axis_name="s")
    shp = jax.ShapeDtypeStruct((B, NSHARD, SHARD), jnp.float32)
    fn = functools.partial(
        pl.kernel,
        mesh=mesh,
        compiler_params=pltpu.CompilerParams(needs_layout_passes=False),
        out_type=[jax.ShapeDtypeStruct((B, NSHARD, SHARD), jnp.int32), shp, shp],
        scratch_types=[
            pltpu.VMEM((SHARD,), jnp.int32),
            pltpu.VMEM((SHARD,), jnp.float32),
            pltpu.VMEM((SHARD,), jnp.float32),
            pltpu.VMEM((SLAB,), jnp.int32),
            pltpu.VMEM((SLAB,), jnp.float32),
            pltpu.VMEM((SLAB,), jnp.float32),
            pltpu.VMEM((4 * HS,), jnp.int32),
            pltpu.VMEM((4 * HS,), jnp.int32),
            pltpu.SemaphoreType.DMA,
        ],
    )(_scatter_body)
    return fn(field.reshape(B, HW), dx.reshape(B, HW), dy.reshape(B, HW),
              minmax)


def _pad2(a, val):
    cval = jnp.full((1, 1), val, a.dtype)
    rows = jnp.concatenate(
        [jnp.broadcast_to(cval, (P, a.shape[1])), a,
         jnp.broadcast_to(cval, (P, a.shape[1]))], axis=0)
    return jnp.concatenate(
        [jnp.broadcast_to(cval, (HP, P)), rows,
         jnp.broadcast_to(cval, (HP, P))], axis=1)


def _shift(a, dr, dc, fill):
    out = a
    if dr:
        z = jnp.full((abs(dr), a.shape[1]), fill, a.dtype)
        out = (jnp.concatenate([z, out[:-dr]], axis=0) if dr > 0
               else jnp.concatenate([out[-dr:], z], axis=0))
    if dc:
        z = jnp.full((a.shape[0], abs(dc)), fill, a.dtype)
        out = (jnp.concatenate([z, out[:, :-dc]], axis=1) if dc > 0
               else jnp.concatenate([out[:, -dc:], z], axis=1))
    return out


def _conv_sep(a):
    v = _G[0] * _shift(a, 1, 0, 0.0) + _G[1] * a + _G[2] * _shift(a, -1, 0, 0.0)
    return _G[0] * _shift(v, 0, 1, 0.0) + _G[1] * v + _G[2] * _shift(v, 0, -1, 0.0)


def _nbr(m):
    return (_shift(m, 1, 0, 0) | _shift(m, -1, 0, 0)
            | _shift(m, 0, 1, 0) | _shift(m, 0, -1, 0))


def _fill_body(win_ref, vx_ref, vy_ref, ox_ref, oy_ref):
    m = _pad2((win_ref[0] < BIG).astype(jnp.int32), 0)
    px = _pad2(vx_ref[0], 0.0)
    py = _pad2(vy_ref[0], 0.0)
    for _ in range(NITER):
        nm = (1 - m) & _nbr(m)
        cx = _conv_sep(px)
        cy = _conv_sep(py)
        cs = _conv_sep(m.astype(jnp.float32))
        hole = nm > 0
        inv = 1.0 / jnp.where(hole, cs, 1.0)
        px = jnp.where(hole, cx * inv, px)
        py = jnp.where(hole, cy * inv, py)
        m = m | nm
    for _ in range(NITER):
        m = m & (1 - (m & _nbr(1 - m)))
    keepm = m > 0
    px = jnp.where(keepm, px, 2.0 * W)[P:-P, P:-P]
    py = jnp.where(keepm, py, 2.0 * H)[P:-P, P:-P]
    gx = lax.broadcasted_iota(jnp.int32, (H, W), 1).astype(jnp.float32)
    gy = lax.broadcasted_iota(jnp.int32, (H, W), 0).astype(jnp.float32)
    ox_ref[0] = gx * (2.0 / (W - 1)) - 1.0 + px * (2.0 / W)
    oy_ref[0] = gy * (2.0 / (H - 1)) - 1.0 + py * (2.0 / H)


def _stage_c(win, vx, vy):
    shp = jax.ShapeDtypeStruct((B, H, W), jnp.float32)
    return pl.pallas_call(
        _fill_body,
        grid=(B,),
        in_specs=[pl.BlockSpec((1, H, W), lambda b: (b, 0, 0))] * 3,
        out_specs=[pl.BlockSpec((1, H, W), lambda b: (b, 0, 0))] * 2,
        out_shape=[shp, shp],
    )(win.reshape(B, H, W), vx.reshape(B, H, W), vy.reshape(B, H, W))


def kernel(src_grid):
    sx = src_grid[..., 0]
    sy = src_grid[..., 1]
    field, dx, dy, minmax = _stage_a(sx, sy)
    win, vx, vy = _stage_b(field, dx, dy, minmax)
    ox, oy = _stage_c(win, vx, vy)
    return jnp.stack([ox, oy], axis=-1)


# speedup vs baseline: 2.3974x; 1.0194x over previous
import functools

import jax
import jax.numpy as jnp
from jax import lax
from jax.experimental import pallas as pl
from jax.experimental.pallas import tpu as pltpu
from jax.experimental.pallas import tpu_sc as plsc

B = 8
HS = 256
H = 512
W = 512
HW = H * W
NITER = 5
P = NITER + 1
HP = H + 2 * P

NSHARD = 8
SHARD = HW // NSHARD
SLAB = 4096
BIG = 2 ** 30

_G = [0.27406862, 0.45186276, 0.27406862]


def _field_body(sx_ref, sy_ref, f_ref, dx_ref, dy_ref, mm_ref):
    sx = sx_ref[0]
    sy = sy_ref[0]
    col = lax.broadcasted_iota(jnp.int32, (HS, HS), 1).astype(jnp.float32)
    row = lax.broadcasted_iota(jnp.int32, (HS, HS), 0).astype(jnp.float32)
    dxs = sx - (col * (2.0 / (HS - 1)) - 1.0)
    dys = sy - (row * (2.0 / (HS - 1)) - 1.0)

    def colsep(a):
        left = jnp.concatenate([a[:, :1], a[:, :-1]], axis=1)
        right = jnp.concatenate([a[:, 1:], a[:, -1:]], axis=1)
        return 0.25 * left + 0.75 * a, 0.75 * a + 0.25 * right

    def rowsep(a):
        up = jnp.concatenate([a[:1], a[:-1]], axis=0)
        dn = jnp.concatenate([a[1:], a[-1:]], axis=0)
        return 0.25 * up + 0.75 * a, 0.75 * a + 0.25 * dn

    def planes(a):
        ce, co = colsep(a)
        p00, p10 = rowsep(ce)
        p01, p11 = rowsep(co)
        return (p00, p01, p10, p11)

    px = planes(dxs)
    py = planes(dys)
    colu = lax.broadcasted_iota(jnp.int32, (HS, HS), 1)
    rowu = lax.broadcasted_iota(jnp.int32, (HS, HS), 0)
    fs = []
    dxo = []
    dyo = []
    for plane in range(4):
        rp, cp = plane >> 1, plane & 1
        dx = px[plane] * (W / 2.0)
        dy = py[plane] * (H / 2.0)
        xpos = (2 * colu + cp).astype(jnp.float32)
        ypos = (2 * rowu + rp).astype(jnp.float32)
        xg = jnp.round(xpos + dx).astype(jnp.int32)
        yg = jnp.round(ypos + dy).astype(jnp.int32)
        oob = (xg < 0) | (yg < 0) | (xg > W - 1) | (yg > H - 1)
        t = jnp.where(oob, -1, yg * W + xg)
        fs.append(t)
        dxo.append(dx)
        dyo.append(dy)
    f_ref[0] = jnp.stack(fs, axis=0)
    dx_ref[0] = jnp.stack(dxo, axis=0)
    dy_ref[0] = jnp.stack(dyo, axis=0)
    mns = [jnp.min(jnp.where(t < 0, HW, t), axis=1) for t in fs]
    mxs = [jnp.max(t, axis=1) for t in fs]
    mm_ref[0] = jnp.stack(
        [jnp.concatenate(mns), jnp.concatenate(mxs)], axis=0)


def _stage_a(sx, sy):
    shp = jax.ShapeDtypeStruct((B, 4, HS, HS), jnp.float32)
    return pl.pallas_call(
        _field_body,
        grid=(B,),
        in_specs=[pl.BlockSpec((1, HS, HS), lambda b: (b, 0, 0))] * 2,
        out_specs=[pl.BlockSpec((1, 4, HS, HS), lambda b: (b, 0, 0, 0))] * 3
        + [pl.BlockSpec((1, 2, 4 * HS), lambda b: (b, 0, 0))],
        out_shape=[jax.ShapeDtypeStruct((B, 4, HS, HS), jnp.int32), shp, shp,
                   jax.ShapeDtypeStruct((B, 2, 4 * HS), jnp.int32)],
    )(sx, sy)


def _scatter_body(f_hbm, dx_hbm, dy_hbm, mm_hbm, win_out, vx_out, vy_out,
                  win_v, vx_v, vy_v, f_s, dx_s, dy_s, mn_s, mx_s, sem):
    nc = 2
    wid = lax.axis_index("s") * nc + lax.axis_index("c")
    lane = lax.iota(jnp.int32, 16)
    big = jnp.full((16,), BIG, jnp.int32)
    zero = jnp.zeros((16,), jnp.float32)

    for rnd in range(2):
        task = wid + 32 * rnd
        b = task // NSHARD
        shard = task % NSHARD
        lo = shard * SHARD

        @pl.loop(0, SHARD // 16, unroll=4)
        def _init(j):
            win_v[pl.ds(j * 16, 16)] = big

        pltpu.sync_copy(mm_hbm.at[b, 0], mn_s)
        pltpu.sync_copy(mm_hbm.at[b, 1], mx_s)

        @pl.loop(0, HW // SLAB)
        def _slab(s):
            mn16 = mn_s[pl.ds(s * 16, 16)]
            mx16 = mx_s[pl.ds(s * 16, 16)]
            rel = (mx16 >= lo) & (mn16 < lo + SHARD)
            cnt = jnp.sum(rel.astype(jnp.int32))

            @pl.when(cnt > 0)
            def _():
                base = s * SLAB
                c1 = pltpu.async_copy(f_hbm.at[b, pl.ds(base, SLAB)], f_s, sem)
                c2 = pltpu.async_copy(dx_hbm.at[b, pl.ds(base, SLAB)], dx_s, sem)
                c3 = pltpu.async_copy(dy_hbm.at[b, pl.ds(base, SLAB)], dy_s, sem)
                c1.wait()
                c2.wait()
                c3.wait()

                @pl.loop(0, SLAB // 16, unroll=4)
                def _vec(v):
                    off = v * 16
                    p0 = base + off
                    t = f_s[pl.ds(off, 16)]
                    cnt_, _last = plsc.scan_count(t)
                    keep = cnt_ == 1
                    m = keep & (t >= lo) & (t < lo + SHARD)
                    plane = p0 >> 16
                    r = (p0 >> 8) & 255
                    k0 = p0 & 255
                    i_base = (r * 1024 + (plane >> 1) * 512 + 2 * k0
                              + (plane & 1))
                    i_vec = i_base + 2 * lane
                    loc = jnp.clip(t - lo, 0, SHARD - 1)
                    cur = plsc.load_gather(win_v, [loc], mask=m)
                    wm = m & (i_vec < cur)
                    plsc.store_scatter(win_v, [loc], i_vec, mask=wm)
                    plsc.store_scatter(vx_v, [loc], -dx_s[pl.ds(off, 16)],
                                       mask=wm)
                    plsc.store_scatter(vy_v, [loc], -dy_s[pl.ds(off, 16)],
                                       mask=wm)

        pltpu.sync_copy(win_v, win_out.at[b, shard])
        pltpu.sync_copy(vx_v, vx_out.at[b, shard])
        pltpu.sync_copy(vy_v, vy_out.at[b, shard])


def _stage_b(field, dx, dy, minmax):
    mesh = plsc.VectorSubcoreMesh(core_axis_name="c", subcore_axis_name="s")
    shp = jax.ShapeDtypeStruct((B, NSHARD, SHARD), jnp.float32)
    fn = functools.partial(
        pl.kernel,
        mesh=mesh,
        compiler_params=pltpu.CompilerParams(needs_layout_passes=False),
        out_type=[jax.ShapeDtypeStruct((B, NSHARD, SHARD), jnp.int32), shp, shp],
        scratch_types=[
            pltpu.VMEM((SHARD,), jnp.int32),
            pltpu.VMEM((SHARD,), jnp.float32),
            pltpu.VMEM((SHARD,), jnp.float32),
            pltpu.VMEM((SLAB,), jnp.int32),
            pltpu.VMEM((SLAB,), jnp.float32),
            pltpu.VMEM((SLAB,), jnp.float32),
            pltpu.VMEM((4 * HS,), jnp.int32),
            pltpu.VMEM((4 * HS,), jnp.int32),
            pltpu.SemaphoreType.DMA,
        ],
    )(_scatter_body)
    return fn(field.reshape(B, HW), dx.reshape(B, HW), dy.reshape(B, HW),
              minmax)


def _pad2(a, val):
    cval = jnp.full((1, 1), val, a.dtype)
    rows = jnp.concatenate(
        [jnp.broadcast_to(cval, (P, a.shape[1])), a,
         jnp.broadcast_to(cval, (P, a.shape[1]))], axis=0)
    return jnp.concatenate(
        [jnp.broadcast_to(cval, (HP, P)), rows,
         jnp.broadcast_to(cval, (HP, P))], axis=1)


def _shift(a, dr, dc, fill):
    out = a
    if dr:
        z = jnp.full((abs(dr), a.shape[1]), fill, a.dtype)
        out = (jnp.concatenate([z, out[:-dr]], axis=0) if dr > 0
               else jnp.concatenate([out[-dr:], z], axis=0))
    if dc:
        z = jnp.full((a.shape[0], abs(dc)), fill, a.dtype)
        out = (jnp.concatenate([z, out[:, :-dc]], axis=1) if dc > 0
               else jnp.concatenate([out[:, -dc:], z], axis=1))
    return out


def _conv_sep(a):
    v = _G[0] * _shift(a, 1, 0, 0.0) + _G[1] * a + _G[2] * _shift(a, -1, 0, 0.0)
    return _G[0] * _shift(v, 0, 1, 0.0) + _G[1] * v + _G[2] * _shift(v, 0, -1, 0.0)


def _nbr(m):
    return (_shift(m, 1, 0, 0) | _shift(m, -1, 0, 0)
            | _shift(m, 0, 1, 0) | _shift(m, 0, -1, 0))


def _fill_body(win_ref, vx_ref, vy_ref, ox_ref, oy_ref):
    hit = win_ref[0] < BIG
    m = _pad2(hit.astype(jnp.int32), 0)
    px = _pad2(jnp.where(hit, vx_ref[0], 0.0), 0.0)
    py = _pad2(jnp.where(hit, vy_ref[0], 0.0), 0.0)
    for _ in range(NITER):
        nm = (1 - m) & _nbr(m)
        cx = _conv_sep(px)
        cy = _conv_sep(py)
        cs = _conv_sep(m.astype(jnp.float32))
        hole = nm > 0
        inv = 1.0 / jnp.where(hole, cs, 1.0)
        px = jnp.where(hole, cx * inv, px)
        py = jnp.where(hole, cy * inv, py)
        m = m | nm
    for _ in range(NITER):
        m = m & (1 - (m & _nbr(1 - m)))
    keepm = m > 0
    px = jnp.where(keepm, px, 2.0 * W)[P:-P, P:-P]
    py = jnp.where(keepm, py, 2.0 * H)[P:-P, P:-P]
    gx = lax.broadcasted_iota(jnp.int32, (H, W), 1).astype(jnp.float32)
    gy = lax.broadcasted_iota(jnp.int32, (H, W), 0).astype(jnp.float32)
    ox_ref[0] = gx * (2.0 / (W - 1)) - 1.0 + px * (2.0 / W)
    oy_ref[0] = gy * (2.0 / (H - 1)) - 1.0 + py * (2.0 / H)


def _stage_c(win, vx, vy):
    shp = jax.ShapeDtypeStruct((B, H, W), jnp.float32)
    return pl.pallas_call(
        _fill_body,
        grid=(B,),
        in_specs=[pl.BlockSpec((1, H, W), lambda b: (b, 0, 0))] * 3,
        out_specs=[pl.BlockSpec((1, H, W), lambda b: (b, 0, 0))] * 2,
        out_shape=[shp, shp],
    )(win.reshape(B, H, W), vx.reshape(B, H, W), vy.reshape(B, H, W))


def kernel(src_grid):
    sx = src_grid[..., 0]
    sy = src_grid[..., 1]
    field, dx, dy, minmax = _stage_a(sx, sy)
    win, vx, vy = _stage_b(field, dx, dy, minmax)
    ox, oy = _stage_c(win, vx, vy)
    return jnp.stack([ox, oy], axis=-1)
